# baseline (speedup 1.0000x reference)
import functools

import jax
import jax.numpy as jnp
from jax.experimental import pallas as pl

TOKENS = 16384
NUM_EXPERTS = 64
TOP_K = 8
TILE = 256


def _gating_kernel(x_ref, w1_ref, b1_ref, w2_ref, b2_ref, w3_ref, b3_ref,
                   topv_ref, topi_ref, probs_ref):
    x = x_ref[...]
    h = jnp.dot(x, w1_ref[...], preferred_element_type=jnp.float32)
    h = jnp.maximum(h + b1_ref[...], 0.0)
    h = jnp.dot(h, w2_ref[...], preferred_element_type=jnp.float32)
    h = jnp.maximum(h + b2_ref[...], 0.0)
    logits = jnp.dot(h, w3_ref[...], preferred_element_type=jnp.float32)
    logits = logits + b3_ref[...]

    m = jnp.max(logits, axis=1, keepdims=True)
    e = jnp.exp(logits - m)
    s = jnp.sum(e, axis=1, keepdims=True)
    probs = e / s
    probs_ref[...] = probs

    workT = probs.T
    rev_iotaT = (63 - jax.lax.broadcasted_iota(jnp.int32, workT.shape, 0)
                 ).astype(jnp.float32)
    vals = []
    ridxs = []
    for _ in range(TOP_K):
        mx = jnp.max(workT, axis=0, keepdims=True)
        matched = workT == mx
        ri = jnp.max(jnp.where(matched, rev_iotaT, -1.0), axis=0, keepdims=True)
        vals.append(mx)
        ridxs.append(ri)
        workT = jnp.where(matched & (rev_iotaT == ri), -1.0, workT)
    top_valsT = jnp.concatenate(vals, axis=0)
    top_idxT = 63.0 - jnp.concatenate(ridxs, axis=0)
    topv_ref[...] = (top_valsT / jnp.sum(top_valsT, axis=0, keepdims=True)).T
    topi_ref[...] = top_idxT.T.astype(jnp.int32)


@jax.jit
def kernel(x, W1, b1, W2, b2, W3, b3):
    grid = (TOKENS // TILE,)
    out_shapes = (
        jax.ShapeDtypeStruct((TOKENS, TOP_K), jnp.float32),
        jax.ShapeDtypeStruct((TOKENS, TOP_K), jnp.int32),
        jax.ShapeDtypeStruct((TOKENS, NUM_EXPERTS), jnp.float32),
    )
    wspec = lambda shape: pl.BlockSpec(shape, lambda i: (0, 0))
    out = pl.pallas_call(
        _gating_kernel,
        grid=grid,
        in_specs=[
            pl.BlockSpec((TILE, 4096), lambda i: (i, 0)),
            wspec((4096, 256)),
            wspec((1, 256)),
            wspec((256, 128)),
            wspec((1, 128)),
            wspec((128, NUM_EXPERTS)),
            wspec((1, NUM_EXPERTS)),
        ],
        out_specs=(
            pl.BlockSpec((TILE, TOP_K), lambda i: (i, 0)),
            pl.BlockSpec((TILE, TOP_K), lambda i: (i, 0)),
            pl.BlockSpec((TILE, NUM_EXPERTS), lambda i: (i, 0)),
        ),
        out_shape=out_shapes,
    )(x, W1, b1.reshape(1, -1), W2, b2.reshape(1, -1), W3, b3.reshape(1, -1))
    return out


# speedup vs baseline: 1.3594x; 1.3594x over previous
import functools

import jax
import jax.numpy as jnp
from jax.experimental import pallas as pl

TOKENS = 16384
NUM_EXPERTS = 64
TOP_K = 8
TILE = 1024


def _gating_kernel(x_ref, w1_ref, b1_ref, w2_ref, b2_ref, w3_ref, b3_ref,
                   topv_ref, topi_ref, probs_ref):
    x = x_ref[...]
    h = jnp.dot(x, w1_ref[...], preferred_element_type=jnp.float32)
    h = jnp.maximum(h + b1_ref[...], 0.0)
    h = jnp.dot(h, w2_ref[...], preferred_element_type=jnp.float32)
    h = jnp.maximum(h + b2_ref[...], 0.0)
    logits = jnp.dot(h, w3_ref[...], preferred_element_type=jnp.float32)
    logits = logits + b3_ref[...]

    m = jnp.max(logits, axis=1, keepdims=True)
    e = jnp.exp(logits - m)
    s = jnp.sum(e, axis=1, keepdims=True)
    probs = e / s
    probs_ref[...] = probs

    workT = probs.T
    rev_iotaT = (63 - jax.lax.broadcasted_iota(jnp.int32, workT.shape, 0)
                 ).astype(jnp.float32)
    vals = []
    ridxs = []
    for _ in range(TOP_K):
        mx = jnp.max(workT, axis=0, keepdims=True)
        matched = workT == mx
        ri = jnp.max(jnp.where(matched, rev_iotaT, -1.0), axis=0, keepdims=True)
        vals.append(mx)
        ridxs.append(ri)
        workT = jnp.where(matched & (rev_iotaT == ri), -1.0, workT)
    top_valsT = jnp.concatenate(vals, axis=0)
    top_idxT = 63.0 - jnp.concatenate(ridxs, axis=0)
    topv_ref[...] = (top_valsT / jnp.sum(top_valsT, axis=0, keepdims=True)).T
    topi_ref[...] = top_idxT.T.astype(jnp.int32)


@jax.jit
def kernel(x, W1, b1, W2, b2, W3, b3):
    grid = (TOKENS // TILE,)
    out_shapes = (
        jax.ShapeDtypeStruct((TOKENS, TOP_K), jnp.float32),
        jax.ShapeDtypeStruct((TOKENS, TOP_K), jnp.int32),
        jax.ShapeDtypeStruct((TOKENS, NUM_EXPERTS), jnp.float32),
    )
    wspec = lambda shape: pl.BlockSpec(shape, lambda i: (0, 0))
    out = pl.pallas_call(
        _gating_kernel,
        grid=grid,
        in_specs=[
            pl.BlockSpec((TILE, 4096), lambda i: (i, 0)),
            wspec((4096, 256)),
            wspec((1, 256)),
            wspec((256, 128)),
            wspec((1, 128)),
            wspec((128, NUM_EXPERTS)),
            wspec((1, NUM_EXPERTS)),
        ],
        out_specs=(
            pl.BlockSpec((TILE, TOP_K), lambda i: (i, 0)),
            pl.BlockSpec((TILE, TOP_K), lambda i: (i, 0)),
            pl.BlockSpec((TILE, NUM_EXPERTS), lambda i: (i, 0)),
        ),
        out_shape=out_shapes,
    )(x, W1, b1.reshape(1, -1), W2, b2.reshape(1, -1), W3, b3.reshape(1, -1))
    return out
